# Initial kernel scaffold; baseline (speedup 1.0000x reference)
import functools

import jax
import jax.numpy as jnp
from jax import lax
from jax.experimental import pallas as pl
from jax.experimental.pallas import tpu as pltpu
from jax.experimental.pallas import tpu_sc as plsc

N_NODES = 10000
N_EDGES = 160000
D = 256

NC = 2
NS = 16
DH = D // NC
EPT = N_EDGES // NS
CH = 80
NCHUNK = EPT // CH
RPT = N_NODES // NS

_sc_mesh = plsc.VectorSubcoreMesh(core_axis_name="c", subcore_axis_name="s")


@functools.partial(
    pl.kernel,
    out_type=(
        jax.ShapeDtypeStruct((N_NODES, D), jnp.float32),
        jax.ShapeDtypeStruct((N_NODES, 16), jnp.float32),
    ),
    mesh=_sc_mesh,
    scratch_types=[
        pltpu.VMEM((CH,), jnp.int32),
        pltpu.VMEM((CH, DH), jnp.float32),
        pltpu.VMEM((CH, 16), jnp.float32),
        pltpu.VMEM_SHARED((N_NODES, DH), jnp.float32),
        pltpu.VMEM_SHARED((N_NODES, 16), jnp.float32),
    ],
)
def _scatter_sum_sc(dest_hbm, ea_hbm, zeros_hbm, ones_hbm,
                    sums_hbm, cnt_hbm, idx_v, ebuf, ones_v, acc, cacc):
    c = lax.axis_index("c")
    s = lax.axis_index("s")

    pltpu.sync_copy(zeros_hbm.at[pl.ds(s * RPT, RPT), :],
                    acc.at[pl.ds(s * RPT, RPT), :])

    @pl.when(c == 0)
    def _():
        pltpu.sync_copy(zeros_hbm.at[pl.ds(s * RPT, RPT), pl.ds(0, 16)],
                        cacc.at[pl.ds(s * RPT, RPT), :])
        pltpu.sync_copy(ones_hbm, ones_v)

    plsc.subcore_barrier()

    ebase = s * EPT

    def chunk_body(k, carry):
        e0 = ebase + k * CH
        pltpu.sync_copy(dest_hbm.at[pl.ds(e0, CH)], idx_v)
        pltpu.sync_copy(ea_hbm.at[pl.ds(e0, CH), pl.ds(c * DH, DH)], ebuf)
        pltpu.sync_copy(ebuf, acc.at[idx_v], add=True)

        @pl.when(c == 0)
        def _():
            pltpu.sync_copy(ones_v, cacc.at[idx_v], add=True)

        return carry

    lax.fori_loop(0, NCHUNK, chunk_body, 0)

    plsc.subcore_barrier()

    pltpu.sync_copy(acc.at[pl.ds(s * RPT, RPT), :],
                    sums_hbm.at[pl.ds(s * RPT, RPT), pl.ds(c * DH, DH)])

    @pl.when(c == 0)
    def _():
        pltpu.sync_copy(cacc.at[pl.ds(s * RPT, RPT), :],
                        cnt_hbm.at[pl.ds(s * RPT, RPT), :])


BLK = 2000


def _mlp_tc(x_ref, sums_ref, cnt_ref, w1x_ref, w1m_ref, w2_ref, w3_ref,
            b1_ref, b2_ref, b3_ref, out_ref):
    cnt = cnt_ref[:, 0:1]
    mean = sums_ref[:] / jnp.maximum(cnt, 1.0)
    h = jnp.dot(x_ref[:], w1x_ref[:], preferred_element_type=jnp.float32)
    h = h + jnp.dot(mean, w1m_ref[:], preferred_element_type=jnp.float32)
    h = h + b1_ref[:]
    h = h * jax.nn.sigmoid(h)
    h = jnp.dot(h, w2_ref[:], preferred_element_type=jnp.float32) + b2_ref[:]
    h = h * jax.nn.sigmoid(h)
    out_ref[:] = (jnp.dot(h, w3_ref[:], preferred_element_type=jnp.float32)
                  + b3_ref[:])


def _mlp_call(x, sums, cnt16, w1x, w1m, w2t, w3t, b1, b2, b3):
    grid = N_NODES // BLK
    row = lambda i: (i, 0)
    fixed = lambda i: (0, 0)
    return pl.pallas_call(
        _mlp_tc,
        grid=(grid,),
        in_specs=[
            pl.BlockSpec((BLK, D), row),
            pl.BlockSpec((BLK, D), row),
            pl.BlockSpec((BLK, 16), row),
            pl.BlockSpec((D, D), fixed),
            pl.BlockSpec((D, D), fixed),
            pl.BlockSpec((D, D), fixed),
            pl.BlockSpec((D, D), fixed),
            pl.BlockSpec((1, D), fixed),
            pl.BlockSpec((1, D), fixed),
            pl.BlockSpec((1, D), fixed),
        ],
        out_specs=pl.BlockSpec((BLK, D), row),
        out_shape=jax.ShapeDtypeStruct((N_NODES, D), jnp.float32),
    )(x, sums, cnt16, w1x, w1m, w2t, w3t, b1, b2, b3)


def kernel(x, dest, edge_attr, W1, b1, W2, b2, W3, b3):
    dest = dest.astype(jnp.int32)
    zeros = jnp.zeros((N_NODES, DH), jnp.float32)
    ones = jnp.ones((CH, 16), jnp.float32)
    sums, cnt16 = _scatter_sum_sc(dest, edge_attr, zeros, ones)
    w1x = W1[:, :D].T
    w1m = W1[:, D:].T
    return _mlp_call(x, sums, cnt16, w1x, w1m, W2.T, W3.T,
                     b1.reshape(1, D), b2.reshape(1, D), b3.reshape(1, D))


# speedup vs baseline: 1.9216x; 1.9216x over previous
import functools

import jax
import jax.numpy as jnp
from jax import lax
from jax.experimental import pallas as pl
from jax.experimental.pallas import tpu as pltpu
from jax.experimental.pallas import tpu_sc as plsc

N_NODES = 10000
N_EDGES = 160000
D = 256

NC = 2
NS = 16
DH = D // NC
PAD_NODES = 10240
RPT = PAD_NODES // NS

CH = 128
NCHUNKS = N_EDGES // CH

_sc_mesh = plsc.VectorSubcoreMesh(core_axis_name="c", subcore_axis_name="s")


@functools.partial(
    pl.kernel,
    out_type=jax.ShapeDtypeStruct((NC, PAD_NODES, DH), jnp.float32),
    mesh=_sc_mesh,
    scratch_types=[
        pltpu.VMEM((CH,), jnp.int32),
        pltpu.VMEM((CH,), jnp.int32),
        pltpu.VMEM((CH, DH), jnp.float32),
        pltpu.SemaphoreType.DMA,
        pltpu.VMEM_SHARED((PAD_NODES, DH), jnp.float32),
    ],
)
def _scatter_sum_sc(dest_hbm, ea2_hbm, eidx_hbm, zeros_hbm, sums_hbm,
                    idx_v, eidx_v, ebuf, sem, acc):
    c = lax.axis_index("c")
    s = lax.axis_index("s")

    pltpu.sync_copy(zeros_hbm, ebuf)
    for j in range(RPT // CH):
        pltpu.sync_copy(ebuf, acc.at[pl.ds(s * RPT + j * CH, CH), :])

    plsc.subcore_barrier()

    nck = 78 + jnp.where(s < NCHUNKS - 78 * NS, 1, 0)

    def chunk_body(k, carry):
        e0 = (k * NS + s) * CH
        pltpu.sync_copy(dest_hbm.at[pl.ds(e0, CH)], idx_v)
        pltpu.sync_copy(eidx_hbm.at[c, 0, pl.ds(e0, CH)], eidx_v)
        pltpu.async_copy(ea2_hbm.at[eidx_v], ebuf, sem).wait()
        pltpu.sync_copy(ebuf, acc.at[idx_v], add=True)
        return carry

    lax.fori_loop(0, nck, chunk_body, 0)

    plsc.subcore_barrier()

    for j in range(RPT // CH):
        pltpu.sync_copy(acc.at[pl.ds(s * RPT + j * CH, CH), :], ebuf)
        pltpu.sync_copy(ebuf, sums_hbm.at[c, pl.ds(s * RPT + j * CH, CH), :])


BLK = 2000


def _mlp_tc(x_ref, sums_ref, cnt_ref, w1x_ref, w1m_ref, w2_ref, w3_ref,
            b1_ref, b2_ref, b3_ref, out_ref):
    cnt = cnt_ref[:, 0:1]
    mean = sums_ref[:] / jnp.maximum(cnt, 1.0)
    h = jnp.dot(x_ref[:], w1x_ref[:], preferred_element_type=jnp.float32)
    h = h + jnp.dot(mean, w1m_ref[:], preferred_element_type=jnp.float32)
    h = h + b1_ref[:]
    h = h * jax.nn.sigmoid(h)
    h = jnp.dot(h, w2_ref[:], preferred_element_type=jnp.float32) + b2_ref[:]
    h = h * jax.nn.sigmoid(h)
    out_ref[:] = (jnp.dot(h, w3_ref[:], preferred_element_type=jnp.float32)
                  + b3_ref[:])


def _mlp_call(x, sums, cnt16, w1x, w1m, w2t, w3t, b1, b2, b3):
    grid = N_NODES // BLK
    row = lambda i: (i, 0)
    fixed = lambda i: (0, 0)
    return pl.pallas_call(
        _mlp_tc,
        grid=(grid,),
        in_specs=[
            pl.BlockSpec((BLK, D), row),
            pl.BlockSpec((BLK, D), row),
            pl.BlockSpec((BLK, 16), row),
            pl.BlockSpec((D, D), fixed),
            pl.BlockSpec((D, D), fixed),
            pl.BlockSpec((D, D), fixed),
            pl.BlockSpec((D, D), fixed),
            pl.BlockSpec((1, D), fixed),
            pl.BlockSpec((1, D), fixed),
            pl.BlockSpec((1, D), fixed),
        ],
        out_specs=pl.BlockSpec((BLK, D), row),
        out_shape=jax.ShapeDtypeStruct((N_NODES, D), jnp.float32),
    )(x, sums, cnt16, w1x, w1m, w2t, w3t, b1, b2, b3)


def kernel(x, dest, edge_attr, W1, b1, W2, b2, W3, b3):
    dest = dest.astype(jnp.int32)
    ea2 = edge_attr.reshape(2 * N_EDGES, DH)
    ar = jnp.arange(N_EDGES, dtype=jnp.int32)
    eidx = jnp.stack([2 * ar, 2 * ar + 1]).reshape(NC, 1, N_EDGES)
    zeros = jnp.zeros((CH, DH), jnp.float32)
    sums2 = _scatter_sum_sc(dest, ea2, eidx, zeros)
    sums = jnp.concatenate([sums2[0, :N_NODES], sums2[1, :N_NODES]], axis=1)
    cnt = jnp.zeros((N_NODES,), jnp.float32).at[dest].add(1.0)
    cnt16 = jnp.broadcast_to(cnt[:, None], (N_NODES, 16))
    w1x = W1[:, :D].T
    w1m = W1[:, D:].T
    return _mlp_call(x, sums, cnt16, w1x, w1m, W2.T, W3.T,
                     b1.reshape(1, D), b2.reshape(1, D), b3.reshape(1, D))


# speedup vs baseline: 2.0210x; 1.0517x over previous
import functools

import jax
import jax.numpy as jnp
from jax import lax
from jax.experimental import pallas as pl
from jax.experimental.pallas import tpu as pltpu
from jax.experimental.pallas import tpu_sc as plsc

N_NODES = 10000
N_EDGES = 160000
D = 256

NC = 2
NS = 16
DH = D // NC
PAD_NODES = 10240
RPT = PAD_NODES // NS

CH = 128
NCHUNKS = N_EDGES // CH

_sc_mesh = plsc.VectorSubcoreMesh(core_axis_name="c", subcore_axis_name="s")


@functools.partial(
    pl.kernel,
    out_type=jax.ShapeDtypeStruct((NC, PAD_NODES, DH), jnp.float32),
    mesh=_sc_mesh,
    scratch_types=[
        pltpu.VMEM((CH,), jnp.int32),
        pltpu.VMEM((CH,), jnp.int32),
        pltpu.VMEM((CH, DH), jnp.float32),
        pltpu.SemaphoreType.DMA,
        pltpu.VMEM_SHARED((PAD_NODES, DH), jnp.float32),
    ],
)
def _scatter_sum_sc(dest_hbm, ea2_hbm, eidx_hbm, zeros_hbm, sums_hbm,
                    idx_v, eidx_v, ebuf, sem, acc):
    c = lax.axis_index("c")
    s = lax.axis_index("s")

    pltpu.sync_copy(zeros_hbm, ebuf)
    for j in range(RPT // CH):
        pltpu.sync_copy(ebuf, acc.at[pl.ds(s * RPT + j * CH, CH), :])

    plsc.subcore_barrier()

    nck = 78 + jnp.where(s < NCHUNKS - 78 * NS, 1, 0)

    def chunk_body(k, carry):
        e0 = (k * NS + s) * CH
        pltpu.sync_copy(dest_hbm.at[pl.ds(e0, CH)], idx_v)
        pltpu.sync_copy(eidx_hbm.at[c, 0, pl.ds(e0, CH)], eidx_v)
        pltpu.async_copy(ea2_hbm.at[eidx_v], ebuf, sem).wait()
        pltpu.sync_copy(ebuf, acc.at[idx_v], add=True)
        return carry

    lax.fori_loop(0, nck, chunk_body, 0)

    plsc.subcore_barrier()

    for j in range(RPT // CH):
        pltpu.sync_copy(acc.at[pl.ds(s * RPT + j * CH, CH), :], ebuf)
        pltpu.sync_copy(ebuf, sums_hbm.at[c, pl.ds(s * RPT + j * CH, CH), :])


@functools.partial(
    pl.kernel,
    out_type=jax.ShapeDtypeStruct((NC, PAD_NODES, DH), jnp.float32),
    mesh=_sc_mesh,
    scratch_types=[
        pltpu.VMEM((CH,), jnp.int32),
        pltpu.VMEM((CH, DH), jnp.float32),
        pltpu.VMEM_SHARED((PAD_NODES, DH), jnp.float32),
    ],
)
def _scatter_cnt_sc(dest_hbm, ones_hbm, zeros_hbm, cnt_hbm, idx_v, ones_v,
                    cacc):
    c = lax.axis_index("c")
    s = lax.axis_index("s")

    pltpu.sync_copy(zeros_hbm, ones_v)
    for j in range(RPT // CH):
        pltpu.sync_copy(ones_v, cacc.at[pl.ds(s * RPT + j * CH, CH), :])
    pltpu.sync_copy(ones_hbm, ones_v)

    plsc.subcore_barrier()

    nck = 39 + jnp.where(s < 625 - 39 * NS, 1, 0)

    def chunk_body(k, carry):
        e0 = (2 * (k * NS + s) + c) * CH
        pltpu.sync_copy(dest_hbm.at[pl.ds(e0, CH)], idx_v)
        pltpu.sync_copy(ones_v, cacc.at[idx_v], add=True)
        return carry

    lax.fori_loop(0, nck, chunk_body, 0)

    plsc.subcore_barrier()

    for j in range(RPT // CH):
        pltpu.sync_copy(cacc.at[pl.ds(s * RPT + j * CH, CH), :], ones_v)
        pltpu.sync_copy(ones_v, cnt_hbm.at[c, pl.ds(s * RPT + j * CH, CH), :])


BLK = 2000


def _mlp_tc(x_ref, sums_ref, cnt_ref, w1x_ref, w1m_ref, w2_ref, w3_ref,
            b1_ref, b2_ref, b3_ref, out_ref):
    cnt = cnt_ref[:, 0:1]
    mean = sums_ref[:] / jnp.maximum(cnt, 1.0)
    h = jnp.dot(x_ref[:], w1x_ref[:], preferred_element_type=jnp.float32)
    h = h + jnp.dot(mean, w1m_ref[:], preferred_element_type=jnp.float32)
    h = h + b1_ref[:]
    h = h * jax.nn.sigmoid(h)
    h = jnp.dot(h, w2_ref[:], preferred_element_type=jnp.float32) + b2_ref[:]
    h = h * jax.nn.sigmoid(h)
    out_ref[:] = (jnp.dot(h, w3_ref[:], preferred_element_type=jnp.float32)
                  + b3_ref[:])


def _mlp_call(x, sums, cnt16, w1x, w1m, w2t, w3t, b1, b2, b3):
    grid = N_NODES // BLK
    row = lambda i: (i, 0)
    fixed = lambda i: (0, 0)
    return pl.pallas_call(
        _mlp_tc,
        grid=(grid,),
        in_specs=[
            pl.BlockSpec((BLK, D), row),
            pl.BlockSpec((BLK, D), row),
            pl.BlockSpec((BLK, 16), row),
            pl.BlockSpec((D, D), fixed),
            pl.BlockSpec((D, D), fixed),
            pl.BlockSpec((D, D), fixed),
            pl.BlockSpec((D, D), fixed),
            pl.BlockSpec((1, D), fixed),
            pl.BlockSpec((1, D), fixed),
            pl.BlockSpec((1, D), fixed),
        ],
        out_specs=pl.BlockSpec((BLK, D), row),
        out_shape=jax.ShapeDtypeStruct((N_NODES, D), jnp.float32),
    )(x, sums, cnt16, w1x, w1m, w2t, w3t, b1, b2, b3)


def kernel(x, dest, edge_attr, W1, b1, W2, b2, W3, b3):
    dest = dest.astype(jnp.int32)
    ea2 = edge_attr.reshape(2 * N_EDGES, DH)
    ar = jnp.arange(N_EDGES, dtype=jnp.int32)
    eidx = jnp.stack([2 * ar, 2 * ar + 1]).reshape(NC, 1, N_EDGES)
    zeros = jnp.zeros((CH, DH), jnp.float32)
    ones = jnp.ones((CH, DH), jnp.float32)
    sums2 = _scatter_sum_sc(dest, ea2, eidx, zeros)
    cnt2 = _scatter_cnt_sc(dest, ones, zeros)
    sums = jnp.concatenate([sums2[0, :N_NODES], sums2[1, :N_NODES]], axis=1)
    cnt16 = (cnt2[0, :N_NODES, :16] + cnt2[1, :N_NODES, :16])
    w1x = W1[:, :D].T
    w1m = W1[:, D:].T
    return _mlp_call(x, sums, cnt16, w1x, w1m, W2.T, W3.T,
                     b1.reshape(1, D), b2.reshape(1, D), b3.reshape(1, D))


# speedup vs baseline: 2.0690x; 1.0238x over previous
import functools

import numpy as np

import jax
import jax.numpy as jnp
from jax import lax
from jax.experimental import pallas as pl
from jax.experimental.pallas import tpu as pltpu
from jax.experimental.pallas import tpu_sc as plsc

N_NODES = 10000
N_EDGES = 160000
D = 256

NC = 2
NS = 16
DH = D // NC
PAD_NODES = 10240
RPT = PAD_NODES // NS

CH = 128
NCHUNKS = N_EDGES // CH

_sc_mesh = plsc.VectorSubcoreMesh(core_axis_name="c", subcore_axis_name="s")


@functools.partial(
    pl.kernel,
    out_type=jax.ShapeDtypeStruct((NC, PAD_NODES, DH), jnp.float32),
    mesh=_sc_mesh,
    scratch_types=[
        pltpu.VMEM((CH,), jnp.int32),
        pltpu.VMEM((CH,), jnp.int32),
        pltpu.VMEM((CH, DH), jnp.float32),
        pltpu.SemaphoreType.DMA,
        pltpu.VMEM_SHARED((PAD_NODES, DH), jnp.float32),
    ],
)
def _scatter_sum_sc(dest_hbm, ea2_hbm, eidx_hbm, zeros_hbm, sums_hbm,
                    idx_v, eidx_v, ebuf, sem, acc):
    c = lax.axis_index("c")
    s = lax.axis_index("s")

    pltpu.sync_copy(zeros_hbm, ebuf)
    for j in range(RPT // CH):
        pltpu.sync_copy(ebuf, acc.at[pl.ds(s * RPT + j * CH, CH), :])

    plsc.subcore_barrier()

    nck = 78 + jnp.where(s < NCHUNKS - 78 * NS, 1, 0)

    def chunk_body(k, carry):
        e0 = (k * NS + s) * CH
        pltpu.sync_copy(dest_hbm.at[pl.ds(e0, CH)], idx_v)
        pltpu.sync_copy(eidx_hbm.at[c, 0, pl.ds(e0, CH)], eidx_v)
        pltpu.async_copy(ea2_hbm.at[eidx_v], ebuf, sem).wait()
        pltpu.sync_copy(ebuf, acc.at[idx_v], add=True)
        return carry

    lax.fori_loop(0, nck, chunk_body, 0)

    plsc.subcore_barrier()

    for j in range(RPT // CH):
        pltpu.sync_copy(acc.at[pl.ds(s * RPT + j * CH, CH), :], ebuf)
        pltpu.sync_copy(ebuf, sums_hbm.at[c, pl.ds(s * RPT + j * CH, CH), :])


@functools.partial(
    pl.kernel,
    out_type=jax.ShapeDtypeStruct((NC, PAD_NODES, DH), jnp.float32),
    mesh=_sc_mesh,
    scratch_types=[
        pltpu.VMEM((CH,), jnp.int32),
        pltpu.VMEM((CH, DH), jnp.float32),
        pltpu.VMEM_SHARED((PAD_NODES, DH), jnp.float32),
    ],
)
def _scatter_cnt_sc(dest_hbm, ones_hbm, zeros_hbm, cnt_hbm, idx_v, ones_v,
                    cacc):
    c = lax.axis_index("c")
    s = lax.axis_index("s")

    pltpu.sync_copy(zeros_hbm, ones_v)
    for j in range(RPT // CH):
        pltpu.sync_copy(ones_v, cacc.at[pl.ds(s * RPT + j * CH, CH), :])
    pltpu.sync_copy(ones_hbm, ones_v)

    plsc.subcore_barrier()

    nck = 39 + jnp.where(s < 625 - 39 * NS, 1, 0)

    def chunk_body(k, carry):
        e0 = (2 * (k * NS + s) + c) * CH
        pltpu.sync_copy(dest_hbm.at[pl.ds(e0, CH)], idx_v)
        pltpu.sync_copy(ones_v, cacc.at[idx_v], add=True)
        return carry

    lax.fori_loop(0, nck, chunk_body, 0)

    plsc.subcore_barrier()

    for j in range(RPT // CH):
        pltpu.sync_copy(cacc.at[pl.ds(s * RPT + j * CH, CH), :], ones_v)
        pltpu.sync_copy(ones_v, cnt_hbm.at[c, pl.ds(s * RPT + j * CH, CH), :])


BLK = 2000

_NT = (((1,), (1,)), ((), ()))


def _mlp_tc(x_ref, sums_ref, cnt_ref, w1_ref, w2_ref, w3_ref,
            b1_ref, b2_ref, b3_ref, out_ref):
    cnt = cnt_ref[0, :, 0:1] + cnt_ref[1, :, 0:1]
    inv = 1.0 / jnp.maximum(cnt, 1.0)
    h = lax.dot_general(x_ref[:], w1_ref[:, :D], _NT,
                        preferred_element_type=jnp.float32)
    h = h + lax.dot_general(sums_ref[0] * inv, w1_ref[:, D:D + DH], _NT,
                            preferred_element_type=jnp.float32)
    h = h + lax.dot_general(sums_ref[1] * inv, w1_ref[:, D + DH:], _NT,
                            preferred_element_type=jnp.float32)
    h = h + b1_ref[:]
    h = h * jax.nn.sigmoid(h)
    h = lax.dot_general(h, w2_ref[:], _NT,
                        preferred_element_type=jnp.float32) + b2_ref[:]
    h = h * jax.nn.sigmoid(h)
    out_ref[:] = lax.dot_general(h, w3_ref[:], _NT,
                                 preferred_element_type=jnp.float32) + b3_ref[:]


def _mlp_call(x, sums2, cnt2, W1, W2, W3, b1, b2, b3):
    grid = N_NODES // BLK
    row = lambda i: (i, 0)
    row3 = lambda i: (0, i, 0)
    fixed = lambda i: (0, 0)
    return pl.pallas_call(
        _mlp_tc,
        grid=(grid,),
        in_specs=[
            pl.BlockSpec((BLK, D), row),
            pl.BlockSpec((NC, BLK, DH), row3),
            pl.BlockSpec((NC, BLK, DH), row3),
            pl.BlockSpec((D, 2 * D), fixed),
            pl.BlockSpec((D, D), fixed),
            pl.BlockSpec((D, D), fixed),
            pl.BlockSpec((1, D), fixed),
            pl.BlockSpec((1, D), fixed),
            pl.BlockSpec((1, D), fixed),
        ],
        out_specs=pl.BlockSpec((BLK, D), row),
        out_shape=jax.ShapeDtypeStruct((N_NODES, D), jnp.float32),
    )(x, sums2, cnt2, W1, W2, W3, b1, b2, b3)


_AR = np.arange(N_EDGES, dtype=np.int32)
_EIDX = np.stack([2 * _AR, 2 * _AR + 1]).reshape(NC, 1, N_EDGES)
_ZEROS = np.zeros((CH, DH), np.float32)
_ONES = np.ones((CH, DH), np.float32)


def kernel(x, dest, edge_attr, W1, b1, W2, b2, W3, b3):
    dest = dest.astype(jnp.int32)
    ea2 = edge_attr.reshape(2 * N_EDGES, DH)
    sums2 = _scatter_sum_sc(dest, ea2, jnp.asarray(_EIDX), jnp.asarray(_ZEROS))
    cnt2 = _scatter_cnt_sc(dest, jnp.asarray(_ONES), jnp.asarray(_ZEROS))
    return _mlp_call(x, sums2, cnt2, W1, W2, W3,
                     b1.reshape(1, D), b2.reshape(1, D), b3.reshape(1, D))


# speedup vs baseline: 2.0856x; 1.0080x over previous
import functools

import numpy as np

import jax
import jax.numpy as jnp
from jax import lax
from jax.experimental import pallas as pl
from jax.experimental.pallas import tpu as pltpu
from jax.experimental.pallas import tpu_sc as plsc

N_NODES = 10000
N_EDGES = 160000
D = 256

NC = 2
NS = 16
DH = D // NC
PAD_NODES = 10240
RPT = PAD_NODES // NS

CH = 128
NCHUNKS = N_EDGES // CH

_sc_mesh = plsc.VectorSubcoreMesh(core_axis_name="c", subcore_axis_name="s")


@functools.partial(
    pl.kernel,
    out_type=[
        jax.ShapeDtypeStruct((NC, PAD_NODES, DH), jnp.float32),
        jax.ShapeDtypeStruct((NC, PAD_NODES, DH), jnp.float32),
    ],
    mesh=_sc_mesh,
    scratch_types=[
        pltpu.VMEM((CH,), jnp.int32),
        pltpu.VMEM((CH,), jnp.int32),
        pltpu.VMEM((CH, DH), jnp.float32),
        pltpu.SemaphoreType.DMA,
        pltpu.VMEM_SHARED((PAD_NODES, DH), jnp.float32),
    ],
)
def _scatter_sc(dest_hbm, ea2_hbm, eidx_hbm, zeros_hbm, ones_hbm,
                sums_hbm, cnt_hbm, idx_v, eidx_v, ebuf, sem, acc):
    c = lax.axis_index("c")
    s = lax.axis_index("s")

    pltpu.sync_copy(zeros_hbm, ebuf)
    for j in range(RPT // CH):
        pltpu.sync_copy(ebuf, acc.at[pl.ds(s * RPT + j * CH, CH), :])

    plsc.subcore_barrier()

    nck = 78 + jnp.where(s < NCHUNKS - 78 * NS, 1, 0)

    def chunk_body(k, carry):
        e0 = (k * NS + s) * CH
        pltpu.sync_copy(dest_hbm.at[pl.ds(e0, CH)], idx_v)
        pltpu.sync_copy(eidx_hbm.at[c, 0, pl.ds(e0, CH)], eidx_v)
        pltpu.async_copy(ea2_hbm.at[eidx_v], ebuf, sem).wait()
        pltpu.sync_copy(ebuf, acc.at[idx_v], add=True)
        return carry

    lax.fori_loop(0, nck, chunk_body, 0)

    plsc.subcore_barrier()

    for j in range(RPT // CH):
        pltpu.sync_copy(acc.at[pl.ds(s * RPT + j * CH, CH), :], ebuf)
        pltpu.sync_copy(ebuf, sums_hbm.at[c, pl.ds(s * RPT + j * CH, CH), :])
    pltpu.sync_copy(zeros_hbm, ebuf)
    for j in range(RPT // CH):
        pltpu.sync_copy(ebuf, acc.at[pl.ds(s * RPT + j * CH, CH), :])
    pltpu.sync_copy(ones_hbm, ebuf)

    plsc.subcore_barrier()

    nck2 = 39 + jnp.where(s < 625 - 39 * NS, 1, 0)

    def cnt_body(k, carry):
        e0 = (2 * (k * NS + s) + c) * CH
        pltpu.sync_copy(dest_hbm.at[pl.ds(e0, CH)], idx_v)
        pltpu.sync_copy(ebuf, acc.at[idx_v], add=True)
        return carry

    lax.fori_loop(0, nck2, cnt_body, 0)

    plsc.subcore_barrier()

    for j in range(RPT // CH):
        pltpu.sync_copy(acc.at[pl.ds(s * RPT + j * CH, CH), :], ebuf)
        pltpu.sync_copy(ebuf, cnt_hbm.at[c, pl.ds(s * RPT + j * CH, CH), :])


BLK = 2000

_NT = (((1,), (1,)), ((), ()))


def _mlp_tc(x_ref, sums_ref, cnt_ref, w1_ref, w2_ref, w3_ref,
            b1_ref, b2_ref, b3_ref, out_ref):
    cnt = cnt_ref[0, :, 0:1] + cnt_ref[1, :, 0:1]
    inv = 1.0 / jnp.maximum(cnt, 1.0)
    h = lax.dot_general(x_ref[:], w1_ref[:, :D], _NT,
                        preferred_element_type=jnp.float32)
    h = h + lax.dot_general(sums_ref[0] * inv, w1_ref[:, D:D + DH], _NT,
                            preferred_element_type=jnp.float32)
    h = h + lax.dot_general(sums_ref[1] * inv, w1_ref[:, D + DH:], _NT,
                            preferred_element_type=jnp.float32)
    h = h + b1_ref[:]
    h = h * jax.nn.sigmoid(h)
    h = lax.dot_general(h, w2_ref[:], _NT,
                        preferred_element_type=jnp.float32) + b2_ref[:]
    h = h * jax.nn.sigmoid(h)
    out_ref[:] = lax.dot_general(h, w3_ref[:], _NT,
                                 preferred_element_type=jnp.float32) + b3_ref[:]


def _mlp_call(x, sums2, cnt2, W1, W2, W3, b1, b2, b3):
    grid = N_NODES // BLK
    row = lambda i: (i, 0)
    row3 = lambda i: (0, i, 0)
    fixed = lambda i: (0, 0)
    return pl.pallas_call(
        _mlp_tc,
        grid=(grid,),
        in_specs=[
            pl.BlockSpec((BLK, D), row),
            pl.BlockSpec((NC, BLK, DH), row3),
            pl.BlockSpec((NC, BLK, DH), row3),
            pl.BlockSpec((D, 2 * D), fixed),
            pl.BlockSpec((D, D), fixed),
            pl.BlockSpec((D, D), fixed),
            pl.BlockSpec((1, D), fixed),
            pl.BlockSpec((1, D), fixed),
            pl.BlockSpec((1, D), fixed),
        ],
        out_specs=pl.BlockSpec((BLK, D), row),
        out_shape=jax.ShapeDtypeStruct((N_NODES, D), jnp.float32),
    )(x, sums2, cnt2, W1, W2, W3, b1, b2, b3)


_AR = np.arange(N_EDGES, dtype=np.int32)
_EIDX = np.stack([2 * _AR, 2 * _AR + 1]).reshape(NC, 1, N_EDGES)
_ZEROS = np.zeros((CH, DH), np.float32)
_ONES = np.ones((CH, DH), np.float32)


def kernel(x, dest, edge_attr, W1, b1, W2, b2, W3, b3):
    dest = dest.astype(jnp.int32)
    ea2 = edge_attr.reshape(2 * N_EDGES, DH)
    sums2, cnt2 = _scatter_sc(dest, ea2, jnp.asarray(_EIDX),
                              jnp.asarray(_ZEROS), jnp.asarray(_ONES))
    return _mlp_call(x, sums2, cnt2, W1, W2, W3,
                     b1.reshape(1, D), b2.reshape(1, D), b3.reshape(1, D))
